# baseline (speedup 1.0000x reference)
import jax
import jax.numpy as jnp
from jax.experimental import pallas as pl
from jax.experimental.pallas import tpu as pltpu
from jax.experimental.pallas import tpu_sc as plsc

_SB = 16
_LANES = 16


def _sc_body(x_vmem, pos_vmem, o_vmem):
    hidden = x_vmem.shape[2]

    @pl.loop(0, _SB)
    def _(i):
        @pl.loop(0, hidden, step=_LANES)
        def _(j):
            o_vmem.at[0, i, pl.ds(j, _LANES)][...] = (
                x_vmem.at[0, i, pl.ds(j, _LANES)][...]
                + pos_vmem.at[i, pl.ds(j, _LANES)][...]
            )


def kernel(x, pos_table):
    batch, seq, hidden = x.shape
    pos = pos_table[:seq]
    mesh = plsc.VectorSubcoreMesh(
        core_axis_name="core", subcore_axis_name="subcore"
    )

    @pl.kernel(out_type=jax.ShapeDtypeStruct(x.shape, x.dtype), mesh=mesh)
    def run(x_hbm, pos_hbm, o_hbm):
        pltpu.emit_pipeline(
            _sc_body,
            grid=(seq // _SB, batch),
            in_specs=[
                pl.BlockSpec((1, _SB, hidden), lambda s, b: (b, s, 0)),
                pl.BlockSpec((_SB, hidden), lambda s, b: (s, 0)),
            ],
            out_specs=[
                pl.BlockSpec((1, _SB, hidden), lambda s, b: (b, s, 0))
            ],
            core_axis_name=("core", "subcore"),
            dimension_semantics=(pltpu.PARALLEL, pltpu.ARBITRARY),
        )(x_hbm, pos_hbm, o_hbm)

    return run(x, pos)


# speedup vs baseline: 4.1550x; 4.1550x over previous
import jax
import jax.numpy as jnp
from jax.experimental import pallas as pl
from jax.experimental.pallas import tpu as pltpu

_SEQ_BLOCK = 1024


def _add_kernel(x_ref, pos_ref, o_ref):
    o_ref[...] = x_ref[...] + pos_ref[...]


def kernel(x, pos_table):
    batch, seq_len, hidden = x.shape
    pos = pos_table[:seq_len]
    sblocks = seq_len // _SEQ_BLOCK

    grid = (sblocks, batch)
    out = pl.pallas_call(
        _add_kernel,
        grid=grid,
        in_specs=[
            pl.BlockSpec((1, _SEQ_BLOCK, hidden), lambda s, b: (b, s, 0)),
            pl.BlockSpec((_SEQ_BLOCK, hidden), lambda s, b: (s, 0)),
        ],
        out_specs=pl.BlockSpec((1, _SEQ_BLOCK, hidden), lambda s, b: (b, s, 0)),
        out_shape=jax.ShapeDtypeStruct((batch, seq_len, hidden), x.dtype),
        compiler_params=pltpu.CompilerParams(
            dimension_semantics=("parallel", "arbitrary"),
        ),
    )(x, pos)
    return out


# speedup vs baseline: 4.3280x; 1.0417x over previous
import jax
import jax.numpy as jnp
from jax.experimental import pallas as pl
from jax.experimental.pallas import tpu as pltpu

_SEQ_BLOCK = 2048


def _add_kernel(x_ref, pos_ref, o_ref):
    o_ref[...] = x_ref[...] + pos_ref[...]


def kernel(x, pos_table):
    batch, seq_len, hidden = x.shape
    pos = pos_table[:seq_len]
    sblocks = seq_len // _SEQ_BLOCK

    grid = (sblocks, batch)
    out = pl.pallas_call(
        _add_kernel,
        grid=grid,
        in_specs=[
            pl.BlockSpec((1, _SEQ_BLOCK, hidden), lambda s, b: (b, s, 0)),
            pl.BlockSpec((_SEQ_BLOCK, hidden), lambda s, b: (s, 0)),
        ],
        out_specs=pl.BlockSpec((1, _SEQ_BLOCK, hidden), lambda s, b: (b, s, 0)),
        out_shape=jax.ShapeDtypeStruct((batch, seq_len, hidden), x.dtype),
        compiler_params=pltpu.CompilerParams(
            dimension_semantics=("parallel", "arbitrary"),
        ),
    )(x, pos)
    return out
